# baseline (speedup 1.0000x reference)
import functools

import jax
import jax.numpy as jnp
from jax import lax
from jax.experimental import pallas as pl
from jax.experimental.pallas import tpu as pltpu, tpu_sc as plsc

F32 = jnp.float32
NC, NS = 2, 16
NW = NC * NS


def _klin(p, x):
    return x @ p["W"].T + p["b"]


def _kbn(x, p, eps=1e-5):
    mu = jnp.mean(x, axis=0)
    var = jnp.var(x, axis=0)
    return (x - mu) / jnp.sqrt(var + eps) * p["g"] + p["b"]


def _ksilu(x):
    return x * jax.nn.sigmoid(x)


def _kmlp(p, x):
    return _ksilu(_kbn(_klin(p["lin"], x), p["norm"]))


def _krbf(x, vmin, vmax, bins):
    centers = jnp.linspace(vmin, vmax, bins, dtype=jnp.float32)
    gamma = 1.0 / ((vmax - vmin) / (bins - 1))
    return jnp.exp(-gamma * (x[:, None] - centers[None, :]) ** 2)


def _sc_edge_body(nchunks, C, rows_per_sub,
                  t1_hbm, t2_hbm, ey_hbm, src_hbm, dst_hbm, zeros_hbm,
                  eout_hbm, s_hbm,
                  sidx, didx, t1v, t2v, eyv, msv, acc, sem1, sem2):
    c = lax.axis_index("c")
    s = lax.axis_index("s")
    wid = s * NC + c
    per_w = nchunks * C

    @pl.when(s == 0)
    def _zero():
        pltpu.sync_copy(zeros_hbm, acc)

    plsc.subcore_barrier()

    def chunk(k, _):
        off = wid * per_w + k * C
        pltpu.sync_copy(src_hbm.at[pl.ds(off, C)], sidx)
        pltpu.sync_copy(dst_hbm.at[pl.ds(off, C)], didx)
        g1 = pltpu.async_copy(t1_hbm.at[sidx], t1v, sem1)
        g2 = pltpu.async_copy(t2_hbm.at[didx], t2v, sem2)
        pltpu.sync_copy(ey_hbm.at[pl.ds(off, C)], eyv)
        g1.wait()
        g2.wait()

        def edge(j, _):
            e = t1v[j, 0:16] + t2v[j] + eyv[j]
            t2v[j] = e
            sig = 1.0 / (1.0 + jnp.exp(-e))
            msv[j, 0:16] = t1v[j, 16:32] * sig
            msv[j, 16:32] = sig
            return _

        lax.fori_loop(0, C, edge, None)
        pltpu.sync_copy(t2v, eout_hbm.at[pl.ds(off, C)])
        pltpu.sync_copy(msv, acc.at[didx], add=True)
        return _

    lax.fori_loop(0, nchunks, chunk, None)
    plsc.subcore_barrier()
    n_nodes = zeros_hbm.shape[0]
    tail = n_nodes - (NS - 1) * rows_per_sub
    r0 = s * rows_per_sub

    @pl.when(s < NS - 1)
    def _flush():
        pltpu.sync_copy(acc.at[pl.ds(r0, rows_per_sub)],
                        s_hbm.at[c].at[pl.ds(r0, rows_per_sub)])

    @pl.when(s == NS - 1)
    def _flush_tail():
        pltpu.sync_copy(acc.at[pl.ds((NS - 1) * rows_per_sub, tail)],
                        s_hbm.at[c].at[pl.ds((NS - 1) * rows_per_sub, tail)])


@functools.partial(jax.jit, static_argnames=("n_nodes", "n_edges"))
def _sc_edge_call(t1, t2, ey, src, dst, zeros, n_nodes, n_edges):
    assert n_edges % NW == 0
    per_w = n_edges // NW
    C = 1000
    assert per_w % C == 0
    nchunks = per_w // C
    rows_per_sub = (n_nodes // NS) // 8 * 8
    mesh = plsc.VectorSubcoreMesh(core_axis_name="c", subcore_axis_name="s")
    body = functools.partial(_sc_edge_body, nchunks, C, rows_per_sub)
    f = pl.kernel(
        body,
        out_type=[jax.ShapeDtypeStruct((n_edges, 16), F32),
                  jax.ShapeDtypeStruct((NC, n_nodes, 32), F32)],
        mesh=mesh,
        scratch_types=[
            pltpu.VMEM((C,), jnp.int32),
            pltpu.VMEM((C,), jnp.int32),
            pltpu.VMEM((C, 32), F32),
            pltpu.VMEM((C, 16), F32),
            pltpu.VMEM((C, 16), F32),
            pltpu.VMEM((C, 32), F32),
            pltpu.VMEM_SHARED((n_nodes, 32), F32),
            pltpu.SemaphoreType.DMA,
            pltpu.SemaphoreType.DMA,
        ],
        compiler_params=pltpu.CompilerParams(use_tc_tiling_on_sc=False),
    )
    return f(t1, t2, ey, src, dst, zeros)


def _rbf_block(h, vmin, vmax, bins):
    step = (vmax - vmin) / (bins - 1)
    centers = (vmin + jax.lax.broadcasted_iota(jnp.int32, (1, bins), 1)
               .astype(F32) * step)
    gamma = 1.0 / step
    return jnp.exp(-gamma * (h[:, None] - centers) ** 2)


def _tc_rbf_stats_body(vmin, vmax, bins, nsteps, n, B,
                       h_ref, w1_ref, o_ref, acc):
    i = pl.program_id(0)

    @pl.when(i == 0)
    def _init():
        acc[...] = jnp.zeros_like(acc)

    t1 = jnp.dot(_rbf_block(h_ref[...], vmin, vmax, bins), w1_ref[...].T,
                 preferred_element_type=F32)
    mask = (i * B + jax.lax.broadcasted_iota(jnp.int32, (B, 1), 0)) < n
    t1 = jnp.where(mask, t1, 0.0)
    acc[0] += jnp.sum(t1, axis=0)
    acc[1] += jnp.sum(t1 * t1, axis=0)

    @pl.when(i == nsteps - 1)
    def _out():
        o_ref[...] = acc[...]


def _tc_rbf_mid_body(vmin, vmax, bins, nsteps, n, B,
                     h_ref, w1_ref, b1_ref, g1_ref, bb1_ref, s1_ref,
                     w2_ref, b2_ref, o_ref, so_ref, acc):
    i = pl.program_id(0)

    @pl.when(i == 0)
    def _init():
        acc[...] = jnp.zeros_like(acc)

    t1 = jnp.dot(_rbf_block(h_ref[...], vmin, vmax, bins), w1_ref[...].T,
                 preferred_element_type=F32) + b1_ref[...]
    mu_nb = s1_ref[0] * (1.0 / n)
    var = s1_ref[1] * (1.0 / n) - mu_nb * mu_nb
    mu = mu_nb + b1_ref[...]
    u = (t1 - mu) * jax.lax.rsqrt(var + 1e-5) * g1_ref[...] + bb1_ref[...]
    u = u * jax.nn.sigmoid(u)
    t2 = jnp.dot(u, w2_ref[...].T, preferred_element_type=F32) + b2_ref[...]
    mask = (i * B + jax.lax.broadcasted_iota(jnp.int32, (B, 1), 0)) < n
    t2m = jnp.where(mask, t2, 0.0)
    acc[0] += jnp.sum(t2m, axis=0)
    acc[1] += jnp.sum(t2m * t2m, axis=0)
    o_ref[...] = t2

    @pl.when(i == nsteps - 1)
    def _out():
        so_ref[...] = acc[...]


def _tc_rbf_mlp2(h, vmin, vmax, bins, p1, p2, B):
    n = h.shape[0]
    npad = -(-n // B) * B
    if npad != n:
        h = jnp.pad(h, (0, npad - n))
    nsteps = npad // B
    w1, b1 = p1["lin"]["W"], p1["lin"]["b"]
    g1, bb1 = p1["norm"]["g"], p1["norm"]["b"]
    w2, b2 = p2["lin"]["W"], p2["lin"]["b"]
    d1 = w1.shape[0]
    stats1 = pl.pallas_call(
        functools.partial(_tc_rbf_stats_body, vmin, vmax, bins, nsteps, n, B),
        grid=(nsteps,),
        in_specs=[pl.BlockSpec((B,), lambda i: (i,)),
                  pl.BlockSpec((d1, bins), lambda i: (0, 0))],
        out_specs=pl.BlockSpec((2, d1), lambda i: (0, 0)),
        out_shape=jax.ShapeDtypeStruct((2, d1), F32),
        scratch_shapes=[pltpu.VMEM((2, d1), F32)],
    )(h, w1)
    t2g, stats2 = pl.pallas_call(
        functools.partial(_tc_rbf_mid_body, vmin, vmax, bins, nsteps, n, B),
        grid=(nsteps,),
        in_specs=[pl.BlockSpec((B,), lambda i: (i,)),
                  pl.BlockSpec((d1, bins), lambda i: (0, 0)),
                  pl.BlockSpec((d1,), lambda i: (0,)),
                  pl.BlockSpec((d1,), lambda i: (0,)),
                  pl.BlockSpec((d1,), lambda i: (0,)),
                  pl.BlockSpec((2, d1), lambda i: (0, 0)),
                  pl.BlockSpec((16, d1), lambda i: (0, 0)),
                  pl.BlockSpec((16,), lambda i: (0,))],
        out_specs=[pl.BlockSpec((B, 16), lambda i: (i, 0)),
                   pl.BlockSpec((2, 16), lambda i: (0, 0))],
        out_shape=[jax.ShapeDtypeStruct((npad, 16), F32),
                   jax.ShapeDtypeStruct((2, 16), F32)],
        scratch_shapes=[pltpu.VMEM((2, 16), F32)],
    )(h, w1, b1, g1, bb1, stats1, w2, b2)
    if npad != n:
        t2g = t2g[:n]
    return t2g, stats2


def _bn_from_stats(x, stats, n, g, b, eps=1e-5):
    mu = stats[0] / n
    var = stats[1] / n - mu * mu
    return (x - mu) * jax.lax.rsqrt(var + eps) * g + b


def _sc_lgcompute_body(nchunks, C,
                       t1_hbm, t2_hbm, ey_hbm, src_hbm, dst_hbm,
                       eout_hbm, msg_hbm, sig_hbm,
                       sidx, didx, t1v, t2v, eyv, msgv, sigv, sem1, sem2):
    c = lax.axis_index("c")
    s = lax.axis_index("s")
    wid = s * NC + c
    per_w = nchunks * C

    def chunk(k, _):
        off = wid * per_w + k * C
        pltpu.sync_copy(src_hbm.at[pl.ds(off, C)], sidx)
        pltpu.sync_copy(dst_hbm.at[pl.ds(off, C)], didx)
        g1 = pltpu.async_copy(t1_hbm.at[sidx], t1v, sem1)
        g2 = pltpu.async_copy(t2_hbm.at[didx], t2v, sem2)
        pltpu.sync_copy(ey_hbm.at[pl.ds(off, C)], eyv)
        g1.wait()
        g2.wait()

        def edge(j, _):
            e = t1v[j, 0:16] + t2v[j] + eyv[j]
            t2v[j] = e
            sig = 1.0 / (1.0 + jnp.exp(-e))
            msgv[j] = t1v[j, 16:32] * sig
            sigv[j] = sig
            return _

        lax.fori_loop(0, C, edge, None)
        pltpu.sync_copy(t2v, eout_hbm.at[pl.ds(off, C)])
        pltpu.sync_copy(msgv, msg_hbm.at[pl.ds(off, C)])
        pltpu.sync_copy(sigv, sig_hbm.at[pl.ds(off, C)])
        return _

    lax.fori_loop(0, nchunks, chunk, None)


def _sc_lgscatter_body(nchunks, C, R, n_seg,
                       msg_hbm, sig_hbm, pidx_hbm, zeros_hbm,
                       s_hbm,
                       didx, msv, acc, sem1):
    c = lax.axis_index("c")
    s = lax.axis_index("s")
    wid = s * NC + c
    per_w = nchunks * C
    half = n_seg // 2
    rps = half // NS

    for t, val_hbm in enumerate((msg_hbm, sig_hbm)):
        for p in range(2):
            @pl.when(s == 0)
            def _zero():
                pltpu.sync_copy(zeros_hbm, acc)

            plsc.subcore_barrier()

            def chunk(k, _):
                off = wid * per_w + k * C
                pltpu.sync_copy(pidx_hbm.at[p].at[pl.ds(off, C)], didx)
                pltpu.sync_copy(val_hbm.at[pl.ds(off, C)], msv)
                pltpu.sync_copy(msv, acc.at[didx], add=True)
                return _

            lax.fori_loop(0, nchunks, chunk, None)
            plsc.subcore_barrier()
            pltpu.sync_copy(
                acc.at[pl.ds(s * rps, rps)],
                s_hbm.at[c].at[t].at[pl.ds(p * half + s * rps, rps)])
            plsc.subcore_barrier()


@functools.partial(jax.jit, static_argnames=("n_seg", "n_edges"))
def _sc_lg_call(t1, t2, ey, src, dst, pidx, zeros_r, n_seg, n_edges):
    per_w = n_edges // NW
    C = 1000
    nchunks = per_w // C
    mesh = plsc.VectorSubcoreMesh(core_axis_name="c", subcore_axis_name="s")
    R = zeros_r.shape[0]
    fc = pl.kernel(
        functools.partial(_sc_lgcompute_body, nchunks, C),
        out_type=[jax.ShapeDtypeStruct((n_edges, 16), F32),
                  jax.ShapeDtypeStruct((n_edges, 16), F32),
                  jax.ShapeDtypeStruct((n_edges, 16), F32)],
        mesh=mesh,
        scratch_types=[
            pltpu.VMEM((C,), jnp.int32),
            pltpu.VMEM((C,), jnp.int32),
            pltpu.VMEM((C, 32), F32),
            pltpu.VMEM((C, 16), F32),
            pltpu.VMEM((C, 16), F32),
            pltpu.VMEM((C, 16), F32),
            pltpu.VMEM((C, 16), F32),
            pltpu.SemaphoreType.DMA,
            pltpu.SemaphoreType.DMA,
        ],
        compiler_params=pltpu.CompilerParams(use_tc_tiling_on_sc=False),
    )
    e_out, msg, sig = fc(t1, t2, ey, src, dst)
    fs = pl.kernel(
        functools.partial(_sc_lgscatter_body, nchunks, C, R, n_seg),
        out_type=jax.ShapeDtypeStruct((NC, 2, n_seg, 16), F32),
        mesh=mesh,
        scratch_types=[
            pltpu.VMEM((C,), jnp.int32),
            pltpu.VMEM((C, 16), F32),
            pltpu.VMEM_SHARED((R, 16), F32),
            pltpu.SemaphoreType.DMA,
        ],
        compiler_params=pltpu.CompilerParams(use_tc_tiling_on_sc=False),
    )
    s_out = fs(msg, sig, pidx, zeros_r)
    return e_out, s_out


def _keggc_lg_sc(p, src, dst, node_feats, edge_feats, n_seg, pidx, zeros_r):
    n_edges = src.shape[0]
    x = _ksilu(_kbn(node_feats, p["norm_nodes"]))
    y = _ksilu(_kbn(edge_feats, p["norm_edges"]))
    t1 = jnp.concatenate([_klin(p["src_gate"], x), _klin(p["dst_update"], x)], axis=1)
    t2 = _klin(p["dst_gate"], x)
    ey = _klin(p["edge_gate"], y)
    e_out, s_out = _sc_lg_call(t1, t2, ey, src, dst, pidx, zeros_r, n_seg, n_edges)
    ssum = s_out[0] + s_out[1]
    h = ssum[0] / (ssum[1] + 1e-6)
    x_out = node_feats + _klin(p["src_update"], x) + h
    y2 = edge_feats + e_out
    return x_out, y2


def _keggc_sc(p, src, dst, node_feats, edge_feats, n_nodes, residual, zeros):
    n_edges = src.shape[0]
    x = _ksilu(_kbn(node_feats, p["norm_nodes"]))
    y = _ksilu(_kbn(edge_feats, p["norm_edges"]))
    t1 = jnp.concatenate([_klin(p["src_gate"], x), _klin(p["dst_update"], x)], axis=1)
    t2 = _klin(p["dst_gate"], x)
    ey = _klin(p["edge_gate"], y)
    e_out, s_out = _sc_edge_call(t1, t2, ey, src, dst, zeros, n_nodes, n_edges)
    ssum = s_out[0] + s_out[1]
    h = ssum[:, 0:16] / (ssum[:, 16:32] + 1e-6)
    x_out = _klin(p["src_update"], x) + h
    y2 = e_out
    if residual:
        x_out = node_feats + x_out
        y2 = edge_feats + y2
    return x_out, y2


def kernel(atom_features, r, lg_h, params, edge_index, lg_edge_index):
    src, dst = edge_index[0], edge_index[1]
    lsrc, ldst = lg_edge_index[0], lg_edge_index[1]
    n_nodes = atom_features.shape[0]
    n_edges = r.shape[0]
    n_lg = lg_h.shape[0]
    zeros_n = jnp.zeros((n_nodes, 32), F32)
    half = n_edges // 2
    zeros_r = jnp.zeros((half + 64, 16), F32)
    spread = half + (jnp.arange(ldst.shape[0], dtype=jnp.int32) % 64)
    pidx = jnp.stack([
        jnp.where((ldst >= p * half) & (ldst < (p + 1) * half), ldst - p * half, spread)
        for p in range(2)])
    t2z, zst2 = _tc_rbf_mlp2(lg_h, -1.0, 1.0, 180,
                             params["angle_mlp1"], params["angle_mlp2"], 8192)
    z = _ksilu(_bn_from_stats(t2z, zst2, n_lg,
                              params["angle_mlp2"]["norm"]["g"],
                              params["angle_mlp2"]["norm"]["b"]))
    bondlength = jnp.linalg.norm(r, axis=1)
    t2y, yst2 = _tc_rbf_mlp2(bondlength, 0.0, 8.0, 40,
                             params["edge_mlp1"], params["edge_mlp2"], 8192)
    y = _ksilu(_bn_from_stats(t2y, yst2, n_edges,
                              params["edge_mlp2"]["norm"]["g"],
                              params["edge_mlp2"]["norm"]["b"]))
    x = _kmlp(params["atom_mlp"], atom_features)
    for lp in params["alignn"]:
        x, m = _keggc_sc(lp["node_update"], src, dst, x, y, n_nodes, True, zeros_n)
        y, z = _keggc_lg_sc(lp["edge_update"], lsrc, ldst, m, z, n_edges, pidx, zeros_r)
    xs = [x]
    ys = [y]
    for gp in params["gcn"]:
        nx, ny = _keggc_sc(gp, src, dst, jnp.concatenate(xs, axis=1),
                           jnp.concatenate(ys, axis=1), n_nodes, False, zeros_n)
        xs.append(nx)
        ys.append(ny)
    x = jnp.concatenate(xs, axis=1)
    h = jnp.mean(x, axis=0, keepdims=True)
    out = _klin(params["fc"], h)
    return jnp.squeeze(out)


# speedup vs baseline: 1.0078x; 1.0078x over previous
import functools

import jax
import jax.numpy as jnp
from jax import lax
from jax.experimental import pallas as pl
from jax.experimental.pallas import tpu as pltpu, tpu_sc as plsc

F32 = jnp.float32
NC, NS = 2, 16
NW = NC * NS


def _klin(p, x):
    return x @ p["W"].T + p["b"]


def _kbn(x, p, eps=1e-5):
    mu = jnp.mean(x, axis=0)
    var = jnp.var(x, axis=0)
    return (x - mu) / jnp.sqrt(var + eps) * p["g"] + p["b"]


def _ksilu(x):
    return x * jax.nn.sigmoid(x)


def _kmlp(p, x):
    return _ksilu(_kbn(_klin(p["lin"], x), p["norm"]))


def _krbf(x, vmin, vmax, bins):
    centers = jnp.linspace(vmin, vmax, bins, dtype=jnp.float32)
    gamma = 1.0 / ((vmax - vmin) / (bins - 1))
    return jnp.exp(-gamma * (x[:, None] - centers[None, :]) ** 2)


def _sc_edge_body(nchunks, C, rows_per_sub,
                  t1_hbm, t2_hbm, ey_hbm, src_hbm, dst_hbm, zeros_hbm,
                  eout_hbm, s_hbm,
                  sidx, didx, t1v, t2v, eyv, msv, acc, sem1, sem2):
    c = lax.axis_index("c")
    s = lax.axis_index("s")
    wid = s * NC + c
    per_w = nchunks * C

    @pl.when(s == 0)
    def _zero():
        pltpu.sync_copy(zeros_hbm, acc)

    plsc.subcore_barrier()

    def chunk(k, _):
        off = wid * per_w + k * C
        pltpu.sync_copy(src_hbm.at[pl.ds(off, C)], sidx)
        pltpu.sync_copy(dst_hbm.at[pl.ds(off, C)], didx)
        g1 = pltpu.async_copy(t1_hbm.at[sidx], t1v, sem1)
        g2 = pltpu.async_copy(t2_hbm.at[didx], t2v, sem2)
        pltpu.sync_copy(ey_hbm.at[pl.ds(off, C)], eyv)
        g1.wait()
        g2.wait()

        def edge(j, _):
            e = t1v[j, 0:16] + t2v[j] + eyv[j]
            t2v[j] = e
            sig = 1.0 / (1.0 + jnp.exp(-e))
            msv[j, 0:16] = t1v[j, 16:32] * sig
            msv[j, 16:32] = sig
            return _

        lax.fori_loop(0, C, edge, None)
        pltpu.sync_copy(t2v, eout_hbm.at[pl.ds(off, C)])
        pltpu.sync_copy(msv, acc.at[didx], add=True)
        return _

    lax.fori_loop(0, nchunks, chunk, None)
    plsc.subcore_barrier()
    n_nodes = zeros_hbm.shape[0]
    tail = n_nodes - (NS - 1) * rows_per_sub
    r0 = s * rows_per_sub

    @pl.when(s < NS - 1)
    def _flush():
        pltpu.sync_copy(acc.at[pl.ds(r0, rows_per_sub)],
                        s_hbm.at[c].at[pl.ds(r0, rows_per_sub)])

    @pl.when(s == NS - 1)
    def _flush_tail():
        pltpu.sync_copy(acc.at[pl.ds((NS - 1) * rows_per_sub, tail)],
                        s_hbm.at[c].at[pl.ds((NS - 1) * rows_per_sub, tail)])


@functools.partial(jax.jit, static_argnames=("n_nodes", "n_edges"))
def _sc_edge_call(t1, t2, ey, src, dst, zeros, n_nodes, n_edges):
    assert n_edges % NW == 0
    per_w = n_edges // NW
    C = 1000
    assert per_w % C == 0
    nchunks = per_w // C
    rows_per_sub = (n_nodes // NS) // 8 * 8
    mesh = plsc.VectorSubcoreMesh(core_axis_name="c", subcore_axis_name="s")
    body = functools.partial(_sc_edge_body, nchunks, C, rows_per_sub)
    f = pl.kernel(
        body,
        out_type=[jax.ShapeDtypeStruct((n_edges, 16), F32),
                  jax.ShapeDtypeStruct((NC, n_nodes, 32), F32)],
        mesh=mesh,
        scratch_types=[
            pltpu.VMEM((C,), jnp.int32),
            pltpu.VMEM((C,), jnp.int32),
            pltpu.VMEM((C, 32), F32),
            pltpu.VMEM((C, 16), F32),
            pltpu.VMEM((C, 16), F32),
            pltpu.VMEM((C, 32), F32),
            pltpu.VMEM_SHARED((n_nodes, 32), F32),
            pltpu.SemaphoreType.DMA,
            pltpu.SemaphoreType.DMA,
        ],
        compiler_params=pltpu.CompilerParams(use_tc_tiling_on_sc=False),
    )
    return f(t1, t2, ey, src, dst, zeros)


def _rbf_block(h, vmin, vmax, bins):
    step = (vmax - vmin) / (bins - 1)
    centers = (vmin + jax.lax.broadcasted_iota(jnp.int32, (1, bins), 1)
               .astype(F32) * step)
    gamma = 1.0 / step
    return jnp.exp(-gamma * (h[:, None] - centers) ** 2)


def _tc_rbf_stats_body(vmin, vmax, bins, nsteps, n, B,
                       h_ref, w1_ref, o_ref, acc):
    i = pl.program_id(0)

    @pl.when(i == 0)
    def _init():
        acc[...] = jnp.zeros_like(acc)

    t1 = jnp.dot(_rbf_block(h_ref[...], vmin, vmax, bins), w1_ref[...].T,
                 preferred_element_type=F32)
    mask = (i * B + jax.lax.broadcasted_iota(jnp.int32, (B, 1), 0)) < n
    t1 = jnp.where(mask, t1, 0.0)
    acc[0] += jnp.sum(t1, axis=0)
    acc[1] += jnp.sum(t1 * t1, axis=0)

    @pl.when(i == nsteps - 1)
    def _out():
        o_ref[...] = acc[...]


def _tc_rbf_mid_body(vmin, vmax, bins, nsteps, n, B,
                     h_ref, w1_ref, b1_ref, g1_ref, bb1_ref, s1_ref,
                     w2_ref, b2_ref, o_ref, so_ref, acc):
    i = pl.program_id(0)

    @pl.when(i == 0)
    def _init():
        acc[...] = jnp.zeros_like(acc)

    t1 = jnp.dot(_rbf_block(h_ref[...], vmin, vmax, bins), w1_ref[...].T,
                 preferred_element_type=F32) + b1_ref[...]
    mu_nb = s1_ref[0] * (1.0 / n)
    var = s1_ref[1] * (1.0 / n) - mu_nb * mu_nb
    mu = mu_nb + b1_ref[...]
    u = (t1 - mu) * jax.lax.rsqrt(var + 1e-5) * g1_ref[...] + bb1_ref[...]
    u = u * jax.nn.sigmoid(u)
    t2 = jnp.dot(u, w2_ref[...].T, preferred_element_type=F32) + b2_ref[...]
    mask = (i * B + jax.lax.broadcasted_iota(jnp.int32, (B, 1), 0)) < n
    t2m = jnp.where(mask, t2, 0.0)
    acc[0] += jnp.sum(t2m, axis=0)
    acc[1] += jnp.sum(t2m * t2m, axis=0)
    o_ref[...] = t2

    @pl.when(i == nsteps - 1)
    def _out():
        so_ref[...] = acc[...]


def _tc_rbf_mlp2(h, vmin, vmax, bins, p1, p2, B):
    n = h.shape[0]
    npad = -(-n // B) * B
    if npad != n:
        h = jnp.pad(h, (0, npad - n))
    nsteps = npad // B
    w1, b1 = p1["lin"]["W"], p1["lin"]["b"]
    g1, bb1 = p1["norm"]["g"], p1["norm"]["b"]
    w2, b2 = p2["lin"]["W"], p2["lin"]["b"]
    d1 = w1.shape[0]
    stats1 = pl.pallas_call(
        functools.partial(_tc_rbf_stats_body, vmin, vmax, bins, nsteps, n, B),
        grid=(nsteps,),
        in_specs=[pl.BlockSpec((B,), lambda i: (i,)),
                  pl.BlockSpec((d1, bins), lambda i: (0, 0))],
        out_specs=pl.BlockSpec((2, d1), lambda i: (0, 0)),
        out_shape=jax.ShapeDtypeStruct((2, d1), F32),
        scratch_shapes=[pltpu.VMEM((2, d1), F32)],
    )(h, w1)
    t2g, stats2 = pl.pallas_call(
        functools.partial(_tc_rbf_mid_body, vmin, vmax, bins, nsteps, n, B),
        grid=(nsteps,),
        in_specs=[pl.BlockSpec((B,), lambda i: (i,)),
                  pl.BlockSpec((d1, bins), lambda i: (0, 0)),
                  pl.BlockSpec((d1,), lambda i: (0,)),
                  pl.BlockSpec((d1,), lambda i: (0,)),
                  pl.BlockSpec((d1,), lambda i: (0,)),
                  pl.BlockSpec((2, d1), lambda i: (0, 0)),
                  pl.BlockSpec((16, d1), lambda i: (0, 0)),
                  pl.BlockSpec((16,), lambda i: (0,))],
        out_specs=[pl.BlockSpec((B, 16), lambda i: (i, 0)),
                   pl.BlockSpec((2, 16), lambda i: (0, 0))],
        out_shape=[jax.ShapeDtypeStruct((npad, 16), F32),
                   jax.ShapeDtypeStruct((2, 16), F32)],
        scratch_shapes=[pltpu.VMEM((2, 16), F32)],
    )(h, w1, b1, g1, bb1, stats1, w2, b2)
    if npad != n:
        t2g = t2g[:n]
    return t2g, stats2


def _bn_from_stats(x, stats, n, g, b, eps=1e-5):
    mu = stats[0] / n
    var = stats[1] / n - mu * mu
    return (x - mu) * jax.lax.rsqrt(var + eps) * g + b


def _sc_lgcompute_body(nchunks, C,
                       t1_hbm, t2_hbm, ey_hbm, src_hbm, dst_hbm,
                       eout_hbm, msg_hbm, sig_hbm,
                       sidx, didx, t1v, t2v, eyv, msgv, sigv, sem1, sem2):
    c = lax.axis_index("c")
    s = lax.axis_index("s")
    wid = s * NC + c
    per_w = nchunks * C

    def chunk(k, _):
        off = wid * per_w + k * C
        pltpu.sync_copy(src_hbm.at[pl.ds(off, C)], sidx)
        pltpu.sync_copy(dst_hbm.at[pl.ds(off, C)], didx)
        g1 = pltpu.async_copy(t1_hbm.at[sidx], t1v, sem1)
        g2 = pltpu.async_copy(t2_hbm.at[didx], t2v, sem2)
        pltpu.sync_copy(ey_hbm.at[pl.ds(off, C)], eyv)
        g1.wait()
        g2.wait()

        def edge(j, _):
            e = t1v[j, 0:16] + t2v[j] + eyv[j]
            t2v[j] = e
            sig = 1.0 / (1.0 + jnp.exp(-e))
            msgv[j] = t1v[j, 16:32] * sig
            sigv[j] = sig
            return _

        lax.fori_loop(0, C, edge, None)
        pltpu.sync_copy(t2v, eout_hbm.at[pl.ds(off, C)])
        pltpu.sync_copy(msgv, msg_hbm.at[pl.ds(off, C)])
        pltpu.sync_copy(sigv, sig_hbm.at[pl.ds(off, C)])
        return _

    lax.fori_loop(0, nchunks, chunk, None)


def _sc_lgscatter_body(nchunks, C, R, n_seg,
                       msg_hbm, sig_hbm, pidx_hbm, zeros_hbm,
                       s_hbm,
                       didx0, didx1, msv0, msv1, acc, sem1):
    c = lax.axis_index("c")
    s = lax.axis_index("s")
    wid = s * NC + c
    per_w = nchunks * C
    half = n_seg // 2
    rps = half // NS
    didxs = (didx0, didx1)
    msvs = (msv0, msv1)

    for t, val_hbm in enumerate((msg_hbm, sig_hbm)):
        for p in range(2):
            @pl.when(s == 0)
            def _zero():
                pltpu.sync_copy(zeros_hbm, acc)

            plsc.subcore_barrier()

            def outer(kk, _):
                for b in range(2):
                    k = kk * 2 + b

                    @pl.when(kk > 0)
                    def _drain():
                        pltpu.make_async_copy(
                            msvs[b], acc.at[didxs[b]], sem1).wait()

                    off = wid * per_w + k * C
                    pltpu.sync_copy(pidx_hbm.at[p].at[pl.ds(off, C)], didxs[b])
                    pltpu.sync_copy(val_hbm.at[pl.ds(off, C)], msvs[b])
                    pltpu.async_copy(msvs[b], acc.at[didxs[b]], sem1, add=True)
                return _

            lax.fori_loop(0, nchunks // 2, outer, None)
            for b in range(2):
                pltpu.make_async_copy(msvs[b], acc.at[didxs[b]], sem1).wait()
            plsc.subcore_barrier()
            pltpu.sync_copy(
                acc.at[pl.ds(s * rps, rps)],
                s_hbm.at[c].at[t].at[pl.ds(p * half + s * rps, rps)])
            plsc.subcore_barrier()


@functools.partial(jax.jit, static_argnames=("n_seg", "n_edges"))
def _sc_lg_call(t1, t2, ey, src, dst, pidx, zeros_r, n_seg, n_edges):
    per_w = n_edges // NW
    C = 1000
    nchunks = per_w // C
    mesh = plsc.VectorSubcoreMesh(core_axis_name="c", subcore_axis_name="s")
    R = zeros_r.shape[0]
    fc = pl.kernel(
        functools.partial(_sc_lgcompute_body, nchunks, C),
        out_type=[jax.ShapeDtypeStruct((n_edges, 16), F32),
                  jax.ShapeDtypeStruct((n_edges, 16), F32),
                  jax.ShapeDtypeStruct((n_edges, 16), F32)],
        mesh=mesh,
        scratch_types=[
            pltpu.VMEM((C,), jnp.int32),
            pltpu.VMEM((C,), jnp.int32),
            pltpu.VMEM((C, 32), F32),
            pltpu.VMEM((C, 16), F32),
            pltpu.VMEM((C, 16), F32),
            pltpu.VMEM((C, 16), F32),
            pltpu.VMEM((C, 16), F32),
            pltpu.SemaphoreType.DMA,
            pltpu.SemaphoreType.DMA,
        ],
        compiler_params=pltpu.CompilerParams(use_tc_tiling_on_sc=False),
    )
    e_out, msg, sig = fc(t1, t2, ey, src, dst)
    fs = pl.kernel(
        functools.partial(_sc_lgscatter_body, nchunks, C, R, n_seg),
        out_type=jax.ShapeDtypeStruct((NC, 2, n_seg, 16), F32),
        mesh=mesh,
        scratch_types=[
            pltpu.VMEM((C,), jnp.int32),
            pltpu.VMEM((C,), jnp.int32),
            pltpu.VMEM((C, 16), F32),
            pltpu.VMEM((C, 16), F32),
            pltpu.VMEM_SHARED((R, 16), F32),
            pltpu.SemaphoreType.DMA,
        ],
        compiler_params=pltpu.CompilerParams(use_tc_tiling_on_sc=False),
    )
    s_out = fs(msg, sig, pidx, zeros_r)
    return e_out, s_out


def _keggc_lg_sc(p, src, dst, node_feats, edge_feats, n_seg, pidx, zeros_r):
    n_edges = src.shape[0]
    x = _ksilu(_kbn(node_feats, p["norm_nodes"]))
    y = _ksilu(_kbn(edge_feats, p["norm_edges"]))
    t1 = jnp.concatenate([_klin(p["src_gate"], x), _klin(p["dst_update"], x)], axis=1)
    t2 = _klin(p["dst_gate"], x)
    ey = _klin(p["edge_gate"], y)
    e_out, s_out = _sc_lg_call(t1, t2, ey, src, dst, pidx, zeros_r, n_seg, n_edges)
    ssum = s_out[0] + s_out[1]
    h = ssum[0] / (ssum[1] + 1e-6)
    x_out = node_feats + _klin(p["src_update"], x) + h
    y2 = edge_feats + e_out
    return x_out, y2


def _keggc_sc(p, src, dst, node_feats, edge_feats, n_nodes, residual, zeros):
    n_edges = src.shape[0]
    x = _ksilu(_kbn(node_feats, p["norm_nodes"]))
    y = _ksilu(_kbn(edge_feats, p["norm_edges"]))
    t1 = jnp.concatenate([_klin(p["src_gate"], x), _klin(p["dst_update"], x)], axis=1)
    t2 = _klin(p["dst_gate"], x)
    ey = _klin(p["edge_gate"], y)
    e_out, s_out = _sc_edge_call(t1, t2, ey, src, dst, zeros, n_nodes, n_edges)
    ssum = s_out[0] + s_out[1]
    h = ssum[:, 0:16] / (ssum[:, 16:32] + 1e-6)
    x_out = _klin(p["src_update"], x) + h
    y2 = e_out
    if residual:
        x_out = node_feats + x_out
        y2 = edge_feats + y2
    return x_out, y2


def kernel(atom_features, r, lg_h, params, edge_index, lg_edge_index):
    src, dst = edge_index[0], edge_index[1]
    lsrc, ldst = lg_edge_index[0], lg_edge_index[1]
    n_nodes = atom_features.shape[0]
    n_edges = r.shape[0]
    n_lg = lg_h.shape[0]
    zeros_n = jnp.zeros((n_nodes, 32), F32)
    half = n_edges // 2
    zeros_r = jnp.zeros((half + 64, 16), F32)
    spread = half + (jnp.arange(ldst.shape[0], dtype=jnp.int32) % 64)
    pidx = jnp.stack([
        jnp.where((ldst >= p * half) & (ldst < (p + 1) * half), ldst - p * half, spread)
        for p in range(2)])
    t2z, zst2 = _tc_rbf_mlp2(lg_h, -1.0, 1.0, 180,
                             params["angle_mlp1"], params["angle_mlp2"], 8192)
    z = _ksilu(_bn_from_stats(t2z, zst2, n_lg,
                              params["angle_mlp2"]["norm"]["g"],
                              params["angle_mlp2"]["norm"]["b"]))
    bondlength = jnp.linalg.norm(r, axis=1)
    t2y, yst2 = _tc_rbf_mlp2(bondlength, 0.0, 8.0, 40,
                             params["edge_mlp1"], params["edge_mlp2"], 8192)
    y = _ksilu(_bn_from_stats(t2y, yst2, n_edges,
                              params["edge_mlp2"]["norm"]["g"],
                              params["edge_mlp2"]["norm"]["b"]))
    x = _kmlp(params["atom_mlp"], atom_features)
    for lp in params["alignn"]:
        x, m = _keggc_sc(lp["node_update"], src, dst, x, y, n_nodes, True, zeros_n)
        y, z = _keggc_lg_sc(lp["edge_update"], lsrc, ldst, m, z, n_edges, pidx, zeros_r)
    xs = [x]
    ys = [y]
    for gp in params["gcn"]:
        nx, ny = _keggc_sc(gp, src, dst, jnp.concatenate(xs, axis=1),
                           jnp.concatenate(ys, axis=1), n_nodes, False, zeros_n)
        xs.append(nx)
        ys.append(ny)
    x = jnp.concatenate(xs, axis=1)
    h = jnp.mean(x, axis=0, keepdims=True)
    out = _klin(params["fc"], h)
    return jnp.squeeze(out)
